# Initial kernel scaffold; baseline (speedup 1.0000x reference)
import jax
import jax.numpy as jnp
from jax.experimental import pallas as pl


def kernel(x, edge_index, batch, W1, b1, W_lin1, b_lin1, W_lin2, b_lin2):
    raise NotImplementedError("write your pallas kernel here")


# speedup vs baseline: 23.8875x; 23.8875x over previous
import functools

import jax
import jax.numpy as jnp
from jax import lax
from jax.experimental import pallas as pl
from jax.experimental.pallas import tpu as pltpu
from jax.experimental.pallas import tpu_sc as plsc

N = 50000
E = 800000
IN_DIM = 10
HID = 64
NQ = 4
QW = HID // NQ
G = 128
OUT_DIM = 2

NC = 2
NS = 16
LW = 125
ER = E // LW
CHUNK_ROWS = 8
KC = CHUNK_ROWS * LW
CW = 16

_sc_mesh = plsc.VectorSubcoreMesh(core_axis_name="c", subcore_axis_name="s")


def _fill_rows(ref, n_rows, width, value):
    def body(i, _):
        for k in range(width // 16):
            ref[i, pl.ds(k * 16, 16)] = jnp.full((16,), value, jnp.float32)
        return 0
    lax.fori_loop(0, n_rows, body, 0)


@functools.partial(
    pl.kernel,
    out_type=jax.ShapeDtypeStruct((NC, N, CW), jnp.float32),
    mesh=_sc_mesh,
    compiler_params=pltpu.CompilerParams(use_tc_tiling_on_sc=False),
    scratch_types=[
        pltpu.VMEM((CHUNK_ROWS, LW), jnp.int32),
        pltpu.VMEM((LW, CW), jnp.float32),
        pltpu.VMEM_SHARED((N, CW), jnp.float32),
    ],
)
def _sc_count(eidx_r, cnt_out, dst_v, ones_v, cnt_sp):
    c = lax.axis_index("c")
    s = lax.axis_index("s")
    wid = s * NC + c

    _fill_rows(ones_v, LW, CW, 0.0)
    nz = N // LW
    for jj in range(nz // NS):
        def _z(j=jj):
            row = (j * NS + s) * LW
            pltpu.sync_copy(ones_v, cnt_sp.at[pl.ds(row, LW)])
        _z()
    _fill_rows(ones_v, LW, CW, 1.0)
    plsc.subcore_barrier()

    rows_per_tile = ER // (NC * NS)
    base = wid * rows_per_tile
    nchunk = rows_per_tile // CHUNK_ROWS

    def body(i, _):
        pltpu.sync_copy(eidx_r.at[1, pl.ds(base + i * CHUNK_ROWS, CHUNK_ROWS)],
                        dst_v)
        for j in range(CHUNK_ROWS):
            pltpu.sync_copy(ones_v, cnt_sp.at[dst_v.at[j]], add=True)
        return 0
    lax.fori_loop(0, nchunk, body, 0)

    plsc.subcore_barrier()

    for jj in range(nz // NS):
        def _o(j=jj):
            row = (j * NS + s) * LW
            pltpu.sync_copy(cnt_sp.at[pl.ds(row, LW)],
                            cnt_out.at[c, pl.ds(row, LW)])
        _o()


@functools.partial(
    pl.kernel,
    out_type=jax.ShapeDtypeStruct((NQ, N, QW), jnp.float32),
    mesh=_sc_mesh,
    compiler_params=pltpu.CompilerParams(use_tc_tiling_on_sc=False),
    scratch_types=[
        pltpu.VMEM((CHUNK_ROWS, LW), jnp.int32),
        pltpu.VMEM((CHUNK_ROWS, LW), jnp.int32),
        pltpu.VMEM((KC, QW), jnp.float32),
        pltpu.VMEM((KC, QW), jnp.float32),
        pltpu.VMEM_SHARED((N, QW), jnp.float32),
        pltpu.SemaphoreType.DMA,
    ],
)
def _sc_scatter(eidx_r, hn4, acc_out, src_v, dst_v, rows_v, zero_v, acc_sp,
                sem):
    c = lax.axis_index("c")
    s = lax.axis_index("s")

    rows_per_tile = ER // NS
    base = s * rows_per_tile
    nchunk = rows_per_tile // CHUNK_ROWS
    nz = N // KC

    _fill_rows(zero_v, KC, QW, 0.0)

    for p in range(NQ // NC):
        q = 2 * c + p

        for jj in range((nz + NS - 1) // NS):
            def _z(j=jj):
                jd = j * NS + s
                @pl.when(jd < nz)
                def _():
                    pltpu.sync_copy(zero_v, acc_sp.at[pl.ds(jd * KC, KC)])
            _z()
        plsc.subcore_barrier()

        def body(i, _):
            off = base + i * CHUNK_ROWS
            pltpu.sync_copy(eidx_r.at[0, pl.ds(off, CHUNK_ROWS)], src_v)
            pltpu.sync_copy(eidx_r.at[1, pl.ds(off, CHUNK_ROWS)], dst_v)
            copies = [
                pltpu.async_copy(hn4.at[q].at[src_v.at[j]],
                                 rows_v.at[pl.ds(j * LW, LW)], sem)
                for j in range(CHUNK_ROWS)
            ]
            for cp in copies:
                cp.wait()
            for j in range(CHUNK_ROWS):
                pltpu.sync_copy(rows_v.at[pl.ds(j * LW, LW)],
                                acc_sp.at[dst_v.at[j]], add=True)
            return 0
        lax.fori_loop(0, nchunk, body, 0)

        plsc.subcore_barrier()

        for jj in range((nz + NS - 1) // NS):
            def _o(j=jj):
                jd = j * NS + s
                @pl.when(jd < nz)
                def _():
                    pltpu.sync_copy(acc_sp.at[pl.ds(jd * KC, KC)],
                                    acc_out.at[q, pl.ds(jd * KC, KC)])
            _o()
        plsc.subcore_barrier()


RB = 2000


def _hn_body(cnt_ref, x_ref, w1_ref, hn_ref):
    deg = cnt_ref[0, :, 0:1] + cnt_ref[1, :, 0:1] + 1.0
    dinv = 1.0 / jnp.sqrt(deg)
    h = jnp.dot(x_ref[...], w1_ref[...], preferred_element_type=jnp.float32)
    hn = h * dinv
    for q in range(NQ):
        hn_ref[q] = hn[:, q * QW:(q + 1) * QW]


def _pool_body(hn_ref, acc_ref, cnt_ref, batch_ref, b1_ref, wl1_ref, bl1_ref,
               wl2_ref, bl2_ref, out_ref, pool_scr):
    i = pl.program_id(0)
    deg = cnt_ref[0, :, 0:1] + cnt_ref[1, :, 0:1] + 1.0
    dinv = 1.0 / jnp.sqrt(deg)
    hcat = jnp.concatenate([hn_ref[q] for q in range(NQ)], axis=1)
    acat = jnp.concatenate([acc_ref[q] for q in range(NQ)], axis=1)
    rows = jnp.maximum((hcat + acat) * dinv + b1_ref[...], 0.0)
    ext = jnp.concatenate([rows, jnp.ones((RB, 1), jnp.float32)], axis=1)
    bid = batch_ref[0]
    gid = lax.broadcasted_iota(jnp.int32, (G, RB), 0)
    onehot_t = (jnp.broadcast_to(bid, (G, RB)) == gid).astype(jnp.float32)
    part = jnp.dot(onehot_t, ext, preferred_element_type=jnp.float32)

    @pl.when(i == 0)
    def _():
        pool_scr[...] = jnp.zeros_like(pool_scr)

    pool_scr[...] += part

    @pl.when(i == (N // RB) - 1)
    def _():
        pe = pool_scr[...]
        pooled = pe[:, :HID] / jnp.maximum(pe[:, HID:HID + 1], 1.0)
        z = jnp.maximum(
            jnp.dot(pooled, wl1_ref[...], preferred_element_type=jnp.float32)
            + bl1_ref[...], 0.0)
        o = (jnp.dot(z, wl2_ref[...], preferred_element_type=jnp.float32)
             + bl2_ref[...])
        m = jnp.max(o, axis=-1, keepdims=True)
        lse = jnp.log(jnp.sum(jnp.exp(o - m), axis=-1, keepdims=True)) + m
        out_ref[...] = o - lse


def kernel(x, edge_index, batch, W1, b1, W_lin1, b_lin1, W_lin2, b_lin2):
    eidx_r = edge_index.reshape(2, ER, LW)

    cnt = _sc_count(eidx_r)

    hn4 = pl.pallas_call(
        _hn_body,
        grid=(N // RB,),
        in_specs=[
            pl.BlockSpec((NC, RB, CW), lambda i: (0, i, 0)),
            pl.BlockSpec((RB, IN_DIM), lambda i: (i, 0)),
            pl.BlockSpec((IN_DIM, HID), lambda i: (0, 0)),
        ],
        out_specs=pl.BlockSpec((NQ, RB, QW), lambda i: (0, i, 0)),
        out_shape=jax.ShapeDtypeStruct((NQ, N, QW), jnp.float32),
    )(cnt, x, W1)

    acc4 = _sc_scatter(eidx_r, hn4)

    batch_r = batch.reshape(N // RB, 1, RB)
    out = pl.pallas_call(
        _pool_body,
        grid=(N // RB,),
        in_specs=[
            pl.BlockSpec((NQ, RB, QW), lambda i: (0, i, 0)),
            pl.BlockSpec((NQ, RB, QW), lambda i: (0, i, 0)),
            pl.BlockSpec((NC, RB, CW), lambda i: (0, i, 0)),
            pl.BlockSpec((1, 1, RB), lambda i: (i, 0, 0)),
            pl.BlockSpec((1, HID), lambda i: (0, 0)),
            pl.BlockSpec((HID, HID), lambda i: (0, 0)),
            pl.BlockSpec((1, HID), lambda i: (0, 0)),
            pl.BlockSpec((HID, OUT_DIM), lambda i: (0, 0)),
            pl.BlockSpec((1, OUT_DIM), lambda i: (0, 0)),
        ],
        out_specs=pl.BlockSpec((G, OUT_DIM), lambda i: (0, 0)),
        out_shape=jax.ShapeDtypeStruct((G, OUT_DIM), jnp.float32),
        scratch_shapes=[pltpu.VMEM((G, HID + 1), jnp.float32)],
    )(hn4, acc4, cnt, batch_r, b1.reshape(1, HID), W_lin1,
      b_lin1.reshape(1, HID), W_lin2, b_lin2.reshape(1, OUT_DIM))
    return out
